# baseline (speedup 1.0000x reference)
import functools

import jax
import jax.numpy as jnp
from jax import lax
from jax.experimental import pallas as pl
from jax.experimental.pallas import tpu as pltpu
from jax.experimental.pallas import tpu_sc as plsc

_CHUNK_TC = 1024
_CS = 128
_NBUF = 4
_SLOTS = 33


def _repack_table(cv):
    nb, _, q, _, _ = cv.shape
    s = _SLOTS
    vq = jnp.pad(cv[:, 0], ((0, 0), (2, 2), (2, 2), (2, 2)))
    vu = {}
    for cz in range(3):
        for cy in range(3):
            for cx in range(3):
                vu[(cz, cy, cx)] = lax.slice(
                    vq, (0, cz, cy, cx),
                    (nb, cz + 2 * s, cy + 2 * s, cx + 2 * s), (1, 2, 2, 2))

    copies = []
    for p in range(8):
        az, ay, ax = (p >> 2) & 1, (p >> 1) & 1, p & 1
        words = [vu[(az + dz, ay + dy, ax + dx)]
                 for dz in range(2) for dy in range(2) for dx in range(2)]
        copies.append(jnp.stack(words, axis=-1))
    tab = jnp.stack(copies, axis=1)
    return tab.reshape(-1)


def _enc_body(quant, freqs_ref, px_ref, py_ref, pz_ref, base_ref, w_ref):
    f = pl.program_id(0)
    fq = freqs_ref[f]
    q = quant
    qf = float(q)
    s = _SLOTS
    cells = s * s * s

    slot = []
    par = []
    w0 = []
    w1 = []
    for p_ref in (px_ref, py_ref, pz_ref):
        ang = p_ref[...] * fq
        s_d, p_d, w0_d, w1_d = [], [], [], []
        for a in range(2):
            t = jnp.sin(ang) if a == 0 else jnp.cos(ang)
            coord = (qf / 2.0) * t + (qf - 1.0) / 2.0
            i0f = jnp.floor(coord)
            frac = coord - i0f
            i0 = i0f.astype(jnp.int32)
            s_d.append((i0 >> 1) + 1)
            p_d.append(i0 & 1)
            w0_d.append(1.0 - frac)
            w1_d.append(frac)
        slot.append(s_d)
        par.append(p_d)
        w0.append(w0_d)
        w1.append(w1_d)

    for k in range(8):
        bx, by, bz = (k >> 2) & 1, (k >> 1) & 1, k & 1
        p = par[2][bz] * 4 + par[1][by] * 2 + par[0][bx]
        cell = (slot[2][bz] * s + slot[1][by]) * s + slot[0][bx]
        base = (((f * 8 + k) * 8 + p) * cells + cell) * 8
        base_ref[k, :, :] = base.reshape(8, _CS)
        wz = (w0[2][bz], w1[2][bz])
        wy = (w0[1][by], w1[1][by])
        wx = (w0[0][bx], w1[0][bx])
        for dz in range(2):
            for dy in range(2):
                wzy = wz[dz] * wy[dy] * 10.0
                for dx in range(2):
                    j = dz * 4 + dy * 2 + dx
                    w_ref[k, j, :, :] = (wzy * wx[dx]).reshape(8, _CS)


def _encode(px, py, pz, freqs, quant, interpret=False):
    n_pad = px.shape[0]
    nf = freqs.shape[0]
    nb = nf * 8
    nch = n_pad // _CS
    grid = (nf, n_pad // _CHUNK_TC)
    ch_per_blk = _CHUNK_TC // _CS
    return pl.pallas_call(
        functools.partial(_enc_body, quant),
        grid=grid,
        in_specs=[
            pl.BlockSpec(memory_space=pltpu.SMEM),
            pl.BlockSpec((_CHUNK_TC,), lambda f, c: (c,)),
            pl.BlockSpec((_CHUNK_TC,), lambda f, c: (c,)),
            pl.BlockSpec((_CHUNK_TC,), lambda f, c: (c,)),
        ],
        out_specs=[
            pl.BlockSpec((8, ch_per_blk, _CS), lambda f, c: (f, c, 0)),
            pl.BlockSpec((8, 8, ch_per_blk, _CS), lambda f, c: (f, 0, c, 0)),
        ],
        out_shape=[
            jax.ShapeDtypeStruct((nb, nch, _CS), jnp.int32),
            jax.ShapeDtypeStruct((nb, 8, nch, _CS), jnp.float32),
        ],
        interpret=interpret,
    )(freqs, px, py, pz)


def _sc_body(nb, n_pad, base_hbm, w_hbm, tab_hbm, out_hbm,
             base_v, wrec_v, idx_v, vals_v, feat_v, csem, gsem, osem):
    wid = lax.axis_index("s") * 2 + lax.axis_index("c")
    b_per_w = nb // 32
    nch = n_pad // _CS

    def rec_copy(b, n, buf):
        pltpu.async_copy(base_hbm.at[b, n], base_v.at[buf], csem.at[buf])
        pltpu.async_copy(w_hbm.at[b, :, n], wrec_v.at[buf], csem.at[buf])

    def rec_wait(b, n, buf):
        pltpu.make_async_copy(base_hbm.at[b, n], base_v.at[buf],
                              csem.at[buf]).wait()
        pltpu.make_async_copy(w_hbm.at[b, :, n], wrec_v.at[buf],
                              csem.at[buf]).wait()

    def fire_gathers(buf):
        for g in range(8):
            basev = base_v[buf, pl.ds(g * 16, 16)]
            for j in range(8):
                idx_v[buf, j, pl.ds(g * 16, 16)] = basev + j
        for j in range(8):
            pltpu.async_copy(tab_hbm.at[idx_v.at[buf, j]], vals_v.at[buf, j],
                             gsem.at[buf])

    def gather_wait(buf):
        for j in range(8):
            pltpu.make_async_copy(tab_hbm.at[idx_v.at[buf, j]],
                                  vals_v.at[buf, j], gsem.at[buf]).wait()

    def compute(buf):
        for g in range(8):
            acc = jnp.zeros((16,), jnp.float32)
            for j in range(8):
                vw = vals_v[buf, j, pl.ds(g * 16, 16)]
                ww = wrec_v[buf, j, pl.ds(g * 16, 16)]
                acc = acc + vw * ww
            feat_v[buf, pl.ds(g * 16, 16)] = acc

    for bi in range(b_per_w):
        b = wid * b_per_w + bi
        rec_copy(b, 0, 0)
        rec_copy(b, 1, 1)

        def outer(i, carry, b=b):
            for sub in range(_NBUF):
                n = i * _NBUF + sub
                bcur = sub
                bprev = (sub - 1) % _NBUF

                @pl.when(n < nch)
                def _():
                    rec_wait(b, n, bcur)
                    fire_gathers(bcur)

                @pl.when((n >= 1) & (n <= nch))
                def _():
                    gather_wait(bprev)

                    @pl.when(n - 1 >= _NBUF)
                    def _():
                        pltpu.make_async_copy(
                            feat_v.at[bprev],
                            out_hbm.at[b, pl.ds(0, _CS)],
                            osem.at[bprev]).wait()

                    compute(bprev)
                    off = pl.multiple_of((n - 1) * _CS, _CS)
                    pltpu.async_copy(feat_v.at[bprev],
                                     out_hbm.at[b, pl.ds(off, _CS)],
                                     osem.at[bprev])

                @pl.when(n + 2 < nch)
                def _():
                    rec_copy(b, n + 2, (sub + 2) % _NBUF)

            return carry

        lax.fori_loop(0, (nch + 2 * _NBUF - 1) // _NBUF, outer, 0)
        for buf in range(_NBUF):
            pltpu.make_async_copy(
                feat_v.at[buf], out_hbm.at[b, pl.ds(0, _CS)],
                osem.at[buf]).wait()


def _sc_gather(base, w, tab, nb, n_pad):
    kfn = functools.partial(
        pl.kernel,
        out_type=jax.ShapeDtypeStruct((nb, n_pad), jnp.float32),
        scratch_types=[
            pltpu.VMEM((_NBUF, _CS), jnp.int32),
            pltpu.VMEM((_NBUF, 8, _CS), jnp.float32),
            pltpu.VMEM((_NBUF, 8, _CS), jnp.int32),
            pltpu.VMEM((_NBUF, 8, _CS), jnp.float32),
            pltpu.VMEM((_NBUF, _CS), jnp.float32),
            pltpu.SemaphoreType.DMA((_NBUF,)),
            pltpu.SemaphoreType.DMA((_NBUF,)),
            pltpu.SemaphoreType.DMA((_NBUF,)),
        ],
        mesh=plsc.VectorSubcoreMesh(core_axis_name="c", subcore_axis_name="s"),
    )(functools.partial(_sc_body, nb, n_pad))
    return kfn(base, w, tab)


def kernel(points, cv, freqs):
    n = points.shape[0]
    nf = freqs.shape[0]
    nb = nf * 8
    quant = cv.shape[-1]
    n_pad = ((n + _CHUNK_TC - 1) // _CHUNK_TC) * _CHUNK_TC

    pts_pad = jnp.pad(points, ((0, n_pad - n), (0, 0)))
    px = pts_pad[:, 0]
    py = pts_pad[:, 1]
    pz = pts_pad[:, 2]

    base, w = _encode(px, py, pz, freqs, quant)
    tab = _repack_table(cv)
    feats = _sc_gather(base, w, tab, nb, n_pad)
    return jnp.concatenate([points, feats[:, :n].T], axis=1)


# speedup vs baseline: 1.2914x; 1.2914x over previous
import functools

import jax
import jax.numpy as jnp
from jax import lax
from jax.experimental import pallas as pl
from jax.experimental.pallas import tpu as pltpu
from jax.experimental.pallas import tpu_sc as plsc

_CHUNK_TC = 1024
_CS = 128
_NBUF = 4


def _repack_table(cv):
    nb, _, q, _, _ = cv.shape
    s = q + 1
    vp = jnp.pad(cv[:, 0], ((0, 0), (1, 1), (1, 1), (1, 1)))
    wins = [vp[:, dz:dz + s, dy:dy + s, dx:dx + s]
            for dz in range(2) for dy in range(2) for dx in range(2)]
    tab = jnp.stack(wins, axis=-1)
    return tab.reshape(-1)


def _enc_body(quant, freqs_ref, px_ref, py_ref, pz_ref, base_ref, w_ref):
    f = pl.program_id(0)
    fq = freqs_ref[f]
    q = quant
    qf = float(q)
    s = q + 1
    cells = s * s * s

    slot = []
    w0 = []
    w1 = []
    for p_ref in (px_ref, py_ref, pz_ref):
        ang = p_ref[...] * fq
        s_d, w0_d, w1_d = [], [], []
        for a in range(2):
            t = jnp.sin(ang) if a == 0 else jnp.cos(ang)
            coord = (qf / 2.0) * t + (qf - 1.0) / 2.0
            i0f = jnp.floor(coord)
            frac = coord - i0f
            s_d.append(i0f.astype(jnp.int32) + 1)
            w0_d.append(1.0 - frac)
            w1_d.append(frac)
        slot.append(s_d)
        w0.append(w0_d)
        w1.append(w1_d)

    for k in range(8):
        bx, by, bz = (k >> 2) & 1, (k >> 1) & 1, k & 1
        cell = (slot[2][bz] * s + slot[1][by]) * s + slot[0][bx]
        base = ((f * 8 + k) * cells + cell) * 8
        base_ref[k, :, :] = base.reshape(8, _CS)
        wz = (w0[2][bz], w1[2][bz])
        wy = (w0[1][by], w1[1][by])
        wx = (w0[0][bx], w1[0][bx])
        for dz in range(2):
            for dy in range(2):
                wzy = wz[dz] * wy[dy] * 10.0
                for dx in range(2):
                    j = dz * 4 + dy * 2 + dx
                    w_ref[k, j, :, :] = (wzy * wx[dx]).reshape(8, _CS)


def _encode(px, py, pz, freqs, quant, interpret=False):
    n_pad = px.shape[0]
    nf = freqs.shape[0]
    nb = nf * 8
    nch = n_pad // _CS
    grid = (nf, n_pad // _CHUNK_TC)
    ch_per_blk = _CHUNK_TC // _CS
    return pl.pallas_call(
        functools.partial(_enc_body, quant),
        grid=grid,
        in_specs=[
            pl.BlockSpec(memory_space=pltpu.SMEM),
            pl.BlockSpec((_CHUNK_TC,), lambda f, c: (c,)),
            pl.BlockSpec((_CHUNK_TC,), lambda f, c: (c,)),
            pl.BlockSpec((_CHUNK_TC,), lambda f, c: (c,)),
        ],
        out_specs=[
            pl.BlockSpec((8, ch_per_blk, _CS), lambda f, c: (f, c, 0)),
            pl.BlockSpec((8, 8, ch_per_blk, _CS), lambda f, c: (f, 0, c, 0)),
        ],
        out_shape=[
            jax.ShapeDtypeStruct((nb, nch, _CS), jnp.int32),
            jax.ShapeDtypeStruct((nb, 8, nch, _CS), jnp.float32),
        ],
        interpret=interpret,
    )(freqs, px, py, pz)


def _sc_body(nb, n_pad, base_hbm, w_hbm, tab_hbm, out_hbm,
             base_v, wrec_v, idx_v0, idx_v1, idx_v2, idx_v3,
             vals_v0, vals_v1, vals_v2, vals_v3, feat_v, csem, gsem, osem):
    idx_bufs = (idx_v0, idx_v1, idx_v2, idx_v3)
    vals_bufs = (vals_v0, vals_v1, vals_v2, vals_v3)
    wid = lax.axis_index("s") * 2 + lax.axis_index("c")
    b_per_w = nb // 32
    nch = n_pad // _CS

    def rec_copy(b, n, buf):
        pltpu.async_copy(base_hbm.at[b, n], base_v.at[buf], csem.at[buf])
        pltpu.async_copy(w_hbm.at[b, :, n], wrec_v.at[buf], csem.at[buf])

    def rec_wait(b, n, buf):
        pltpu.make_async_copy(base_hbm.at[b, n], base_v.at[buf],
                              csem.at[buf]).wait()
        pltpu.make_async_copy(w_hbm.at[b, :, n], wrec_v.at[buf],
                              csem.at[buf]).wait()

    def fire_gathers(buf):
        idx_v = idx_bufs[buf]
        for g in range(8):
            basev = base_v[buf, pl.ds(g * 16, 16)]
            for j in range(8):
                idx_v[pl.ds(j * _CS + g * 16, 16)] = basev + j
        pltpu.async_copy(tab_hbm.at[idx_v], vals_bufs[buf], gsem.at[buf])

    def gather_wait(buf):
        pltpu.make_async_copy(tab_hbm.at[idx_bufs[buf]],
                              vals_bufs[buf], gsem.at[buf]).wait()

    def compute(buf):
        for g in range(8):
            acc = jnp.zeros((16,), jnp.float32)
            for j in range(8):
                vw = vals_bufs[buf][pl.ds(j * _CS + g * 16, 16)]
                ww = wrec_v[buf, j, pl.ds(g * 16, 16)]
                acc = acc + vw * ww
            feat_v[buf, pl.ds(g * 16, 16)] = acc

    for bi in range(b_per_w):
        b = wid * b_per_w + bi
        rec_copy(b, 0, 0)
        rec_copy(b, 1, 1)

        def outer(i, carry, b=b):
            for sub in range(_NBUF):
                n = i * _NBUF + sub
                bcur = sub
                bprev = (sub - 1) % _NBUF

                @pl.when(n < nch)
                def _():
                    rec_wait(b, n, bcur)
                    fire_gathers(bcur)

                @pl.when((n >= 1) & (n <= nch))
                def _():
                    gather_wait(bprev)

                    @pl.when(n - 1 >= _NBUF)
                    def _():
                        pltpu.make_async_copy(
                            feat_v.at[bprev],
                            out_hbm.at[b, pl.ds(0, _CS)],
                            osem.at[bprev]).wait()

                    compute(bprev)
                    off = pl.multiple_of((n - 1) * _CS, _CS)
                    pltpu.async_copy(feat_v.at[bprev],
                                     out_hbm.at[b, pl.ds(off, _CS)],
                                     osem.at[bprev])

                @pl.when(n + 2 < nch)
                def _():
                    rec_copy(b, n + 2, (sub + 2) % _NBUF)

            return carry

        lax.fori_loop(0, (nch + 2 * _NBUF - 1) // _NBUF, outer, 0)
        for buf in range(_NBUF):
            pltpu.make_async_copy(
                feat_v.at[buf], out_hbm.at[b, pl.ds(0, _CS)],
                osem.at[buf]).wait()


def _sc_gather(base, w, tab, nb, n_pad):
    kfn = functools.partial(
        pl.kernel,
        out_type=jax.ShapeDtypeStruct((nb, n_pad), jnp.float32),
        scratch_types=[
            pltpu.VMEM((_NBUF, _CS), jnp.int32),
            pltpu.VMEM((_NBUF, 8, _CS), jnp.float32),
            pltpu.VMEM((8 * _CS,), jnp.int32),
            pltpu.VMEM((8 * _CS,), jnp.int32),
            pltpu.VMEM((8 * _CS,), jnp.int32),
            pltpu.VMEM((8 * _CS,), jnp.int32),
            pltpu.VMEM((8 * _CS,), jnp.float32),
            pltpu.VMEM((8 * _CS,), jnp.float32),
            pltpu.VMEM((8 * _CS,), jnp.float32),
            pltpu.VMEM((8 * _CS,), jnp.float32),
            pltpu.VMEM((_NBUF, _CS), jnp.float32),
            pltpu.SemaphoreType.DMA((_NBUF,)),
            pltpu.SemaphoreType.DMA((_NBUF,)),
            pltpu.SemaphoreType.DMA((_NBUF,)),
        ],
        mesh=plsc.VectorSubcoreMesh(core_axis_name="c", subcore_axis_name="s"),
    )(functools.partial(_sc_body, nb, n_pad))
    return kfn(base, w, tab)


def kernel(points, cv, freqs):
    n = points.shape[0]
    nf = freqs.shape[0]
    nb = nf * 8
    quant = cv.shape[-1]
    n_pad = ((n + _CHUNK_TC - 1) // _CHUNK_TC) * _CHUNK_TC

    pts_pad = jnp.pad(points, ((0, n_pad - n), (0, 0)))
    px = pts_pad[:, 0]
    py = pts_pad[:, 1]
    pz = pts_pad[:, 2]

    base, w = _encode(px, py, pz, freqs, quant)
    tab = _repack_table(cv)
    feats = _sc_gather(base, w, tab, nb, n_pad)
    return jnp.concatenate([points, feats[:, :n].T], axis=1)


# speedup vs baseline: 1.2941x; 1.0021x over previous
import functools

import jax
import jax.numpy as jnp
from jax import lax
from jax.experimental import pallas as pl
from jax.experimental.pallas import tpu as pltpu
from jax.experimental.pallas import tpu_sc as plsc

_CHUNK_TC = 2048
_CS = 128
_NBUF = 4


def _repack_table(cv):
    nb, _, q, _, _ = cv.shape
    s = q + 1
    sx = q + 2
    vp = jnp.pad(cv[:, 0], ((0, 0), (1, 1), (1, 1), (1, 1)))
    wins = [vp[:, dz:dz + s, dy:dy + s, :sx]
            for dz in range(2) for dy in range(2)]
    tab = jnp.stack(wins, axis=-1)
    return tab.reshape(-1)


def _enc_body(quant, freqs_ref, px_ref, py_ref, pz_ref, base_ref, w_ref):
    f = pl.program_id(0)
    fq = freqs_ref[f]
    q = quant
    qf = float(q)
    s = q + 1
    sx = q + 2

    slot = []
    w0 = []
    w1 = []
    for p_ref in (px_ref, py_ref, pz_ref):
        ang = p_ref[...] * fq
        s_d, w0_d, w1_d = [], [], []
        for a in range(2):
            t = jnp.sin(ang) if a == 0 else jnp.cos(ang)
            coord = (qf / 2.0) * t + (qf - 1.0) / 2.0
            i0f = jnp.floor(coord)
            frac = coord - i0f
            s_d.append(i0f.astype(jnp.int32) + 1)
            w0_d.append(1.0 - frac)
            w1_d.append(frac)
        slot.append(s_d)
        w0.append(w0_d)
        w1.append(w1_d)

    ch_per_blk = _CHUNK_TC // _CS
    for k in range(8):
        bx, by, bz = (k >> 2) & 1, (k >> 1) & 1, k & 1
        cell = ((slot[2][bz] * s + slot[1][by]) * sx + slot[0][bx]) * 4
        base = (f * 8 + k) * (s * s * sx * 4) + cell
        base_ref[k, :, :] = base.reshape(ch_per_blk, _CS)
        wz = (w0[2][bz], w1[2][bz])
        wy = (w0[1][by], w1[1][by])
        wx = (w0[0][bx], w1[0][bx])
        for dx in range(2):
            for dz in range(2):
                for dy in range(2):
                    j = dx * 4 + dz * 2 + dy
                    wv = wz[dz] * wy[dy] * wx[dx] * 10.0
                    w_ref[k, j, :, :] = wv.reshape(ch_per_blk, _CS)


def _encode(px, py, pz, freqs, quant, interpret=False):
    n_pad = px.shape[0]
    nf = freqs.shape[0]
    nb = nf * 8
    nch = n_pad // _CS
    grid = (nf, n_pad // _CHUNK_TC)
    ch_per_blk = _CHUNK_TC // _CS
    return pl.pallas_call(
        functools.partial(_enc_body, quant),
        grid=grid,
        in_specs=[
            pl.BlockSpec(memory_space=pltpu.SMEM),
            pl.BlockSpec((_CHUNK_TC,), lambda f, c: (c,)),
            pl.BlockSpec((_CHUNK_TC,), lambda f, c: (c,)),
            pl.BlockSpec((_CHUNK_TC,), lambda f, c: (c,)),
        ],
        out_specs=[
            pl.BlockSpec((8, ch_per_blk, _CS), lambda f, c: (f, c, 0)),
            pl.BlockSpec((8, 8, ch_per_blk, _CS), lambda f, c: (f, 0, c, 0)),
        ],
        out_shape=[
            jax.ShapeDtypeStruct((nb, nch, _CS), jnp.int32),
            jax.ShapeDtypeStruct((nb, 8, nch, _CS), jnp.float32),
        ],
        interpret=interpret,
    )(freqs, px, py, pz)


def _sc_body(nb, n_pad, base_hbm, w_hbm, tab_hbm, out_hbm,
             base_v, wrec_v, idx_v0, idx_v1, idx_v2, idx_v3,
             vals_v0, vals_v1, vals_v2, vals_v3, feat_v, csem, gsem, osem):
    idx_bufs = (idx_v0, idx_v1, idx_v2, idx_v3)
    vals_bufs = (vals_v0, vals_v1, vals_v2, vals_v3)
    wid = lax.axis_index("s") * 2 + lax.axis_index("c")
    b_per_w = nb // 32
    nch = n_pad // _CS

    def rec_copy(b, n, buf):
        pltpu.async_copy(base_hbm.at[b, n], base_v.at[buf], csem.at[buf])
        pltpu.async_copy(w_hbm.at[b, :, n], wrec_v.at[buf], csem.at[buf])

    def rec_wait(b, n, buf):
        pltpu.make_async_copy(base_hbm.at[b, n], base_v.at[buf],
                              csem.at[buf]).wait()
        pltpu.make_async_copy(w_hbm.at[b, :, n], wrec_v.at[buf],
                              csem.at[buf]).wait()

    def fire_gathers(buf):
        idx_v = idx_bufs[buf]
        for g in range(_CS // 16):
            basev = base_v[buf, pl.ds(g * 16, 16)]
            for j in range(8):
                idx_v[pl.ds(j * _CS + g * 16, 16)] = basev + j
        pltpu.async_copy(tab_hbm.at[idx_v], vals_bufs[buf], gsem.at[buf])

    def gather_wait(buf):
        pltpu.make_async_copy(tab_hbm.at[idx_bufs[buf]],
                              vals_bufs[buf], gsem.at[buf]).wait()

    def compute(buf):
        for g in range(_CS // 16):
            acc = jnp.zeros((16,), jnp.float32)
            for j in range(8):
                vw = vals_bufs[buf][pl.ds(j * _CS + g * 16, 16)]
                ww = wrec_v[buf, j, pl.ds(g * 16, 16)]
                acc = acc + vw * ww
            feat_v[buf, pl.ds(g * 16, 16)] = acc

    for bi in range(b_per_w):
        b = wid * b_per_w + bi
        rec_copy(b, 0, 0)
        rec_copy(b, 1, 1)

        def outer(i, carry, b=b):
            for sub in range(_NBUF):
                n = i * _NBUF + sub
                bcur = sub
                bprev = (sub - 1) % _NBUF

                @pl.when(n < nch)
                def _():
                    rec_wait(b, n, bcur)
                    fire_gathers(bcur)

                @pl.when((n >= 1) & (n <= nch))
                def _():
                    gather_wait(bprev)

                    @pl.when(n - 1 >= _NBUF)
                    def _():
                        pltpu.make_async_copy(
                            feat_v.at[bprev],
                            out_hbm.at[b, pl.ds(0, _CS)],
                            osem.at[bprev]).wait()

                    compute(bprev)
                    off = pl.multiple_of((n - 1) * _CS, _CS)
                    pltpu.async_copy(feat_v.at[bprev],
                                     out_hbm.at[b, pl.ds(off, _CS)],
                                     osem.at[bprev])

                @pl.when(n + 2 < nch)
                def _():
                    rec_copy(b, n + 2, (sub + 2) % _NBUF)

            return carry

        lax.fori_loop(0, (nch + 2 * _NBUF - 1) // _NBUF, outer, 0)
        for buf in range(_NBUF):
            pltpu.make_async_copy(
                feat_v.at[buf], out_hbm.at[b, pl.ds(0, _CS)],
                osem.at[buf]).wait()


def _sc_gather(base, w, tab, nb, n_pad):
    kfn = functools.partial(
        pl.kernel,
        out_type=jax.ShapeDtypeStruct((nb, n_pad), jnp.float32),
        scratch_types=[
            pltpu.VMEM((_NBUF, _CS), jnp.int32),
            pltpu.VMEM((_NBUF, 8, _CS), jnp.float32),
            pltpu.VMEM((8 * _CS,), jnp.int32),
            pltpu.VMEM((8 * _CS,), jnp.int32),
            pltpu.VMEM((8 * _CS,), jnp.int32),
            pltpu.VMEM((8 * _CS,), jnp.int32),
            pltpu.VMEM((8 * _CS,), jnp.float32),
            pltpu.VMEM((8 * _CS,), jnp.float32),
            pltpu.VMEM((8 * _CS,), jnp.float32),
            pltpu.VMEM((8 * _CS,), jnp.float32),
            pltpu.VMEM((_NBUF, _CS), jnp.float32),
            pltpu.SemaphoreType.DMA((_NBUF,)),
            pltpu.SemaphoreType.DMA((_NBUF,)),
            pltpu.SemaphoreType.DMA((_NBUF,)),
        ],
        mesh=plsc.VectorSubcoreMesh(core_axis_name="c", subcore_axis_name="s"),
    )(functools.partial(_sc_body, nb, n_pad))
    return kfn(base, w, tab)


def kernel(points, cv, freqs):
    n = points.shape[0]
    nf = freqs.shape[0]
    nb = nf * 8
    quant = cv.shape[-1]
    n_pad = ((n + _CHUNK_TC - 1) // _CHUNK_TC) * _CHUNK_TC

    pts_pad = jnp.pad(points, ((0, n_pad - n), (0, 0)))
    px = pts_pad[:, 0]
    py = pts_pad[:, 1]
    pz = pts_pad[:, 2]

    base, w = _encode(px, py, pz, freqs, quant)
    tab = _repack_table(cv)
    feats = _sc_gather(base, w, tab, nb, n_pad)
    return jnp.concatenate([points, feats[:, :n].T], axis=1)
